# baseline (speedup 1.0000x reference)
import functools

import jax
import jax.numpy as jnp
import numpy as np
from jax import lax
from jax.experimental import pallas as pl
from jax.experimental.pallas import tpu as pltpu
from jax.experimental.pallas import tpu_sc as plsc

_NUM_FEATURES = 26
_ROWS_PER_FEATURE = 100000
_CHUNK = 128
_SEG = 2000


@functools.cache
def _build_transpose(dim, rows, nw):
    nseg = rows // _SEG
    n_iter = (nseg + nw - 1) // nw
    mesh = plsc.VectorSubcoreMesh(core_axis_name="c", subcore_axis_name="s")

    @functools.partial(
        pl.kernel,
        out_type=jax.ShapeDtypeStruct((rows, dim), jnp.float32),
        mesh=mesh,
        compiler_params=pltpu.CompilerParams(
            use_tc_tiling_on_sc=False, needs_layout_passes=False
        ),
        scratch_types=[
            pltpu.VMEM((dim, _SEG), jnp.float32),
            pltpu.VMEM((_SEG, dim), jnp.float32),
        ],
    )
    def k(src_hbm, dst_hbm, in_v, out_v):
        wid = lax.axis_index("s") * 2 + lax.axis_index("c")
        lanes = lax.iota(jnp.int32, 16)

        def seg_body(i, carry):
            s = i * nw + wid

            @pl.when(s < nseg)
            def _():
                pltpu.sync_copy(src_hbm.at[:, pl.ds(s * _SEG, _SEG)], in_v)

                def r_body(r, c2):
                    out_v.at[r][...] = plsc.load_gather(
                        in_v, [lanes, jnp.full((16,), 0, jnp.int32) + r]
                    )
                    return c2

                lax.fori_loop(0, _SEG, r_body, 0, unroll=8)
                pltpu.sync_copy(out_v, dst_hbm.at[pl.ds(s * _SEG, _SEG)])

            return carry

        lax.fori_loop(0, n_iter, seg_body, 0)

    return k


@functools.cache
def _build_lookup(batch, feat, dim, rows, nw):
    rows_per_w = batch // nw
    n_chunks = rows_per_w // _CHUNK
    chf = _CHUNK * feat
    mesh = plsc.VectorSubcoreMesh(core_axis_name="c", subcore_axis_name="s")

    @functools.partial(
        pl.kernel,
        out_type=jax.ShapeDtypeStruct((batch, dim), jnp.float32),
        mesh=mesh,
        compiler_params=pltpu.CompilerParams(use_tc_tiling_on_sc=False),
        scratch_types=[
            pltpu.VMEM((chf,), jnp.int32),
            pltpu.VMEM((chf,), jnp.int32),
            pltpu.VMEM((chf, dim), jnp.float32),
            pltpu.VMEM((_CHUNK, dim), jnp.float32),
            pltpu.SemaphoreType.DMA,
        ],
    )
    def k(x_hbm, offs_hbm, table_hbm, out_hbm, idx_v, offs_v, rows_v, acc_v, sem):
        wid = lax.axis_index("s") * 2 + lax.axis_index("c")
        base = wid * rows_per_w
        pltpu.sync_copy(offs_hbm, offs_v)

        def chunk_body(c, carry):
            cb = base + c * _CHUNK
            pltpu.sync_copy(x_hbm.at[pl.ds(cb * feat, chf)], idx_v)

            def add_body(i, carry2):
                s = i * 16
                idx_v[pl.ds(s, 16)] = idx_v[pl.ds(s, 16)] + offs_v[pl.ds(s, 16)]
                return carry2

            lax.fori_loop(0, chf // 16, add_body, 0, unroll=8)

            pltpu.async_copy(table_hbm.at[idx_v], rows_v, sem).wait()

            def sum_rows(b, carry3):
                a = rows_v.at[b * feat][...]
                for j in range(1, feat):
                    a = a + rows_v.at[b * feat + j][...]
                acc_v.at[b][...] = a
                return carry3

            lax.fori_loop(0, _CHUNK, sum_rows, 0)
            pltpu.sync_copy(acc_v, out_hbm.at[pl.ds(cb, _CHUNK)])
            return carry

        lax.fori_loop(0, n_chunks, chunk_body, 0)

    return k


def kernel(x, table):
    batch, feat = x.shape
    rows, dim = table.shape
    info = plsc.get_sparse_core_info()
    nw = info.num_cores * info.num_subcores
    offsets = np.arange(feat, dtype=np.int32) * _ROWS_PER_FEATURE
    offs_rep = jnp.asarray(np.tile(offsets, _CHUNK))
    x_flat = x.reshape(-1).astype(jnp.int32)
    table_rm = _build_transpose(dim, rows, nw)(table.T)
    return _build_lookup(batch, feat, dim, rows, nw)(x_flat, offs_rep, table_rm)


# speedup vs baseline: 1.1300x; 1.1300x over previous
import functools

import jax
import jax.numpy as jnp
import numpy as np
from jax import lax
from jax.experimental import pallas as pl
from jax.experimental.pallas import tpu as pltpu
from jax.experimental.pallas import tpu_sc as plsc

_NUM_FEATURES = 26
_ROWS_PER_FEATURE = 100000
_CHUNK = 128


@functools.cache
def _build_lookup(batch, feat, dim, rows, nw):
    rows_per_w = batch // nw
    n_chunks = rows_per_w // _CHUNK
    chf = _CHUNK * feat
    mesh = plsc.VectorSubcoreMesh(core_axis_name="c", subcore_axis_name="s")

    @functools.partial(
        pl.kernel,
        out_type=jax.ShapeDtypeStruct((batch, dim), jnp.float32),
        mesh=mesh,
        compiler_params=pltpu.CompilerParams(
            use_tc_tiling_on_sc=False, needs_layout_passes=False
        ),
        scratch_types=[
            pltpu.VMEM((chf,), jnp.int32),
            pltpu.VMEM((chf,), jnp.int32),
            pltpu.VMEM((dim, chf), jnp.float32),
            pltpu.VMEM((dim, _CHUNK), jnp.float32),
            pltpu.VMEM((_CHUNK, dim), jnp.float32),
            pltpu.SemaphoreType.DMA,
        ],
    )
    def k(x_hbm, table_hbm, out_hbm, xs_v, idx_v, vals_v, accT_v, acc_v, sem):
        wid = lax.axis_index("s") * 2 + lax.axis_index("c")
        base = wid * rows_per_w
        lanes = lax.iota(jnp.int32, 16)

        def chunk_body(c, carry):
            cb = base + c * _CHUNK
            pltpu.sync_copy(x_hbm.at[pl.ds(cb * feat, chf)], xs_v)

            for f in range(feat):
                off_f = jnp.int32(f * _ROWS_PER_FEATURE)
                for t in range(_CHUNK // 16):
                    g = lanes * feat + (t * 16 * feat + f)
                    v = plsc.load_gather(xs_v, [g]) + off_f
                    idx_v[pl.ds(f * _CHUNK + t * 16, 16)] = v

            descs = []
            for d in range(dim):
                descs.append(
                    pltpu.async_copy(
                        table_hbm.at[d].at[idx_v], vals_v.at[d], sem
                    )
                )
            for dsc in descs:
                dsc.wait()

            for d in range(dim):
                for t in range(_CHUNK // 16):
                    a = vals_v.at[d][pl.ds(t * 16, 16)]
                    for f in range(1, feat):
                        a = a + vals_v.at[d][pl.ds(f * _CHUNK + t * 16, 16)]
                    accT_v.at[d][pl.ds(t * 16, 16)] = a

            def tr_body(b, c3):
                acc_v.at[b][...] = plsc.load_gather(
                    accT_v, [lanes, jnp.full((16,), 0, jnp.int32) + b]
                )
                return c3

            lax.fori_loop(0, _CHUNK, tr_body, 0, unroll=8)
            pltpu.sync_copy(acc_v, out_hbm.at[pl.ds(cb, _CHUNK)])
            return carry

        lax.fori_loop(0, n_chunks, chunk_body, 0)

    return k


def kernel(x, table):
    batch, feat = x.shape
    rows, dim = table.shape
    info = plsc.get_sparse_core_info()
    nw = info.num_cores * info.num_subcores
    x_flat = x.reshape(-1).astype(jnp.int32)
    tableT = table.T
    return _build_lookup(batch, feat, dim, rows, nw)(x_flat, tableT)


# speedup vs baseline: 3.3238x; 2.9415x over previous
import functools

import jax
import jax.numpy as jnp
import numpy as np
from jax import lax
from jax.experimental import pallas as pl
from jax.experimental.pallas import tpu as pltpu
from jax.experimental.pallas import tpu_sc as plsc

_NUM_FEATURES = 26
_ROWS_PER_FEATURE = 100000
_CHUNK = 128
_SEG = 1024


@functools.cache
def _build_transpose(dim, rows, nw):
    nseg = (rows // _SEG)
    rows_main = nseg * _SEG
    tail = rows - rows_main
    n_iter = (nseg + nw - 1) // nw
    tps = _SEG // 128
    out_rows = rows * dim // 128
    tail_out = tail * dim // 128
    mesh = plsc.VectorSubcoreMesh(core_axis_name="c", subcore_axis_name="s")

    @functools.partial(
        pl.kernel,
        out_type=jax.ShapeDtypeStruct((out_rows, 128), jnp.float32),
        mesh=mesh,
        compiler_params=pltpu.CompilerParams(
            use_tc_tiling_on_sc=True, needs_layout_passes=False
        ),
        scratch_types=[
            pltpu.VMEM((2 * tps, 8, 128), jnp.float32),
            pltpu.VMEM((_SEG * dim // 128, 128), jnp.float32),
            pltpu.SemaphoreType.DMA,
        ],
    )
    def k(src_hbm, tail_hbm, dst_hbm, in_v, out_v, sem):
        wid = lax.axis_index("s") * 2 + lax.axis_index("c")
        lanes = lax.iota(jnp.int32, 16)
        idx_d = lanes & 7
        plane8 = (lanes >> 3) * tps

        def seg_body(i, carry):
            s = i * nw + wid

            @pl.when(s < nseg)
            def _():
                c0 = s * _SEG
                descs = []
                for h in range(2):
                    for j in range(tps):
                        descs.append(
                            pltpu.async_copy(
                                src_hbm.at[h, :, pl.ds(c0 + j * 128, 128)],
                                in_v.at[h * tps + j],
                                sem,
                            )
                        )
                for dsc in descs:
                    dsc.wait()

                def q_body(q, c2):
                    j = q >> 4
                    l0 = (q & 15) * 8
                    idx_t = plane8 + j
                    for u in range(8):
                        v = plsc.load_gather(
                            in_v,
                            [idx_t, idx_d, jnp.full((16,), 0, jnp.int32) + (l0 + u)],
                        )
                        out_v.at[q][pl.ds(u * dim, dim)] = v
                    return c2

                lax.fori_loop(0, _SEG * dim // 128, q_body, 0)
                pltpu.sync_copy(
                    out_v, dst_hbm.at[pl.ds(s * (_SEG * dim // 128), _SEG * dim // 128)]
                )

            return carry

        lax.fori_loop(0, n_iter, seg_body, 0)

        if tail:
            @pl.when(wid == nw - 1)
            def _():
                pltpu.sync_copy(tail_hbm, out_v.at[pl.ds(0, tail_out)])
                pltpu.sync_copy(
                    out_v.at[pl.ds(0, tail_out)],
                    dst_hbm.at[pl.ds(out_rows - tail_out, tail_out)],
                )

    return k


@functools.cache
def _build_lookup(batch, feat, dim, rows, nw):
    rows_per_w = batch // nw
    n_chunks = rows_per_w // _CHUNK
    chf = _CHUNK * feat
    mesh = plsc.VectorSubcoreMesh(core_axis_name="c", subcore_axis_name="s")

    @functools.partial(
        pl.kernel,
        out_type=jax.ShapeDtypeStruct((batch, dim), jnp.float32),
        mesh=mesh,
        compiler_params=pltpu.CompilerParams(use_tc_tiling_on_sc=False),
        scratch_types=[
            pltpu.VMEM((chf,), jnp.int32),
            pltpu.VMEM((chf,), jnp.int32),
            pltpu.VMEM((chf, dim), jnp.float32),
            pltpu.VMEM((_CHUNK, dim), jnp.float32),
            pltpu.SemaphoreType.DMA,
        ],
    )
    def k(x_hbm, offs_hbm, table_hbm, out_hbm, idx_v, offs_v, rows_v, acc_v, sem):
        wid = lax.axis_index("s") * 2 + lax.axis_index("c")
        base = wid * rows_per_w
        pltpu.sync_copy(offs_hbm, offs_v)

        def chunk_body(c, carry):
            cb = base + c * _CHUNK
            pltpu.sync_copy(x_hbm.at[pl.ds(cb * feat, chf)], idx_v)

            def add_body(i, carry2):
                s = i * 16
                idx_v[pl.ds(s, 16)] = idx_v[pl.ds(s, 16)] + offs_v[pl.ds(s, 16)]
                return carry2

            lax.fori_loop(0, chf // 16, add_body, 0, unroll=8)

            pltpu.async_copy(table_hbm.at[idx_v], rows_v, sem).wait()

            def sum_rows(b, carry3):
                a = rows_v.at[b * feat][...]
                for j in range(1, feat):
                    a = a + rows_v.at[b * feat + j][...]
                acc_v.at[b][...] = a
                return carry3

            lax.fori_loop(0, _CHUNK, sum_rows, 0)
            pltpu.sync_copy(acc_v, out_hbm.at[pl.ds(cb, _CHUNK)])
            return carry

        lax.fori_loop(0, n_chunks, chunk_body, 0)

    return k


def kernel(x, table):
    batch, feat = x.shape
    rows, dim = table.shape
    info = plsc.get_sparse_core_info()
    nw = info.num_cores * info.num_subcores
    rows_main = (rows // _SEG) * _SEG
    offsets = np.arange(feat, dtype=np.int32) * _ROWS_PER_FEATURE
    offs_rep = jnp.asarray(np.tile(offsets, _CHUNK))
    x_flat = x.reshape(-1).astype(jnp.int32)
    tableT3 = table.T.reshape(2, dim // 2, rows)
    tail_packed = table[rows_main:].reshape(-1, 128)
    packed = _build_transpose(dim, rows, nw)(tableT3, tail_packed)
    table_rm = packed.reshape(rows, dim)
    return _build_lookup(batch, feat, dim, rows, nw)(x_flat, offs_rep, table_rm)


# speedup vs baseline: 7.6170x; 2.2916x over previous
import functools

import jax
import jax.numpy as jnp
import numpy as np
from jax import lax
from jax.experimental import pallas as pl
from jax.experimental.pallas import tpu as pltpu
from jax.experimental.pallas import tpu_sc as plsc

_NUM_FEATURES = 26
_ROWS_PER_FEATURE = 100000
_CHUNK = 128
_SEG = 1024


@functools.cache
def _build_transpose(dim, rows, nw):
    nseg = (rows // _SEG)
    rows_main = nseg * _SEG
    tail = rows - rows_main
    n_iter = (nseg + nw - 1) // nw
    tps = _SEG // 128
    out_rows = rows * dim // 128
    tail_out = tail * dim // 128
    mesh = plsc.VectorSubcoreMesh(core_axis_name="c", subcore_axis_name="s")

    @functools.partial(
        pl.kernel,
        out_type=jax.ShapeDtypeStruct((out_rows, 128), jnp.float32),
        mesh=mesh,
        compiler_params=pltpu.CompilerParams(
            use_tc_tiling_on_sc=True, needs_layout_passes=False
        ),
        scratch_types=[
            pltpu.VMEM((2, 2 * tps, 8, 128), jnp.float32),
            pltpu.VMEM((_SEG * dim // 128, 128), jnp.float32),
            pltpu.SemaphoreType.DMA,
            pltpu.SemaphoreType.DMA,
        ],
    )
    def k(src_hbm, tail_hbm, dst_hbm, in_v, out_v, sem0, sem1):
        wid = lax.axis_index("s") * 2 + lax.axis_index("c")
        lanes = lax.iota(jnp.int32, 16)
        rowc = lanes >> 3
        colc = (lanes & 7) * dim

        def fire(s, buf, sem):
            c0 = s * _SEG
            for h in range(2):
                for j in range(tps):
                    pltpu.async_copy(
                        src_hbm.at[h, :, pl.ds(c0 + j * 128, 128)],
                        in_v.at[buf, h * tps + j],
                        sem,
                    )

        def drain(s, buf, sem):
            c0 = s * _SEG
            for h in range(2):
                for j in range(tps):
                    pltpu.make_async_copy(
                        src_hbm.at[h, :, pl.ds(c0 + j * 128, 128)],
                        in_v.at[buf, h * tps + j],
                        sem,
                    ).wait()

        def process(s, buf, sem):
            drain(s, buf, sem)

            def t_body(t, c2):
                h = t >> 3
                j = t & 7
                jb = j * 16
                for d in range(8):
                    col_idx = colc + (h * 8 + d)
                    for lb in range(8):
                        v = in_v[buf, t, d, pl.ds(lb * 16, 16)]
                        row_idx = rowc + (jb + lb * 2)
                        plsc.store_scatter(out_v, [row_idx, col_idx], v)
                return c2

            lax.fori_loop(0, 2 * tps, t_body, 0)
            pltpu.sync_copy(
                out_v, dst_hbm.at[pl.ds(s * (_SEG * dim // 128), _SEG * dim // 128)]
            )

        @pl.when(wid < nseg)
        def _():
            fire(wid, 0, sem0)

        def seg_body(i, carry):
            sa = (2 * i) * nw + wid
            sb = sa + nw
            sc = sa + 2 * nw

            @pl.when(sb < nseg)
            def _():
                fire(sb, 1, sem1)

            @pl.when(sa < nseg)
            def _():
                process(sa, 0, sem0)

            @pl.when(sc < nseg)
            def _():
                fire(sc, 0, sem0)

            @pl.when(sb < nseg)
            def _():
                process(sb, 1, sem1)

            return carry

        lax.fori_loop(0, (n_iter + 1) // 2, seg_body, 0)

        if tail:
            @pl.when(wid == nw - 1)
            def _():
                pltpu.sync_copy(tail_hbm, out_v.at[pl.ds(0, tail_out)])
                pltpu.sync_copy(
                    out_v.at[pl.ds(0, tail_out)],
                    dst_hbm.at[pl.ds(out_rows - tail_out, tail_out)],
                )

    return k


@functools.cache
def _build_lookup(batch, feat, dim, rows, nw):
    rows_per_w = batch // nw
    n_chunks = rows_per_w // _CHUNK
    chf = _CHUNK * feat
    mesh = plsc.VectorSubcoreMesh(core_axis_name="c", subcore_axis_name="s")

    @functools.partial(
        pl.kernel,
        out_type=jax.ShapeDtypeStruct((batch, dim), jnp.float32),
        mesh=mesh,
        compiler_params=pltpu.CompilerParams(use_tc_tiling_on_sc=False),
        scratch_types=[
            pltpu.VMEM((chf,), jnp.int32),
            pltpu.VMEM((chf,), jnp.int32),
            pltpu.VMEM((chf, dim), jnp.float32),
            pltpu.VMEM((_CHUNK, dim), jnp.float32),
            pltpu.SemaphoreType.DMA,
        ],
    )
    def k(x_hbm, offs_hbm, table_hbm, out_hbm, idx_v, offs_v, rows_v, acc_v, sem):
        wid = lax.axis_index("s") * 2 + lax.axis_index("c")
        base = wid * rows_per_w
        pltpu.sync_copy(offs_hbm, offs_v)

        def chunk_body(c, carry):
            cb = base + c * _CHUNK
            pltpu.sync_copy(x_hbm.at[pl.ds(cb * feat, chf)], idx_v)

            def add_body(i, carry2):
                s = i * 16
                idx_v[pl.ds(s, 16)] = idx_v[pl.ds(s, 16)] + offs_v[pl.ds(s, 16)]
                return carry2

            lax.fori_loop(0, chf // 16, add_body, 0, unroll=8)

            pltpu.async_copy(table_hbm.at[idx_v], rows_v, sem).wait()

            def sum_rows(b, carry3):
                a = rows_v.at[b * feat][...]
                for j in range(1, feat):
                    a = a + rows_v.at[b * feat + j][...]
                acc_v.at[b][...] = a
                return carry3

            lax.fori_loop(0, _CHUNK, sum_rows, 0)
            pltpu.sync_copy(acc_v, out_hbm.at[pl.ds(cb, _CHUNK)])
            return carry

        lax.fori_loop(0, n_chunks, chunk_body, 0)

    return k


def kernel(x, table):
    batch, feat = x.shape
    rows, dim = table.shape
    info = plsc.get_sparse_core_info()
    nw = info.num_cores * info.num_subcores
    rows_main = (rows // _SEG) * _SEG
    offsets = np.arange(feat, dtype=np.int32) * _ROWS_PER_FEATURE
    offs_rep = jnp.asarray(np.tile(offsets, _CHUNK))
    x_flat = x.reshape(-1).astype(jnp.int32)
    tableT3 = table.T.reshape(2, dim // 2, rows)
    tail_packed = table[rows_main:].reshape(-1, 128)
    packed = _build_transpose(dim, rows, nw)(tableT3, tail_packed)
    table_rm = packed.reshape(rows, dim)
    return _build_lookup(batch, feat, dim, rows, nw)(x_flat, offs_rep, table_rm)


# speedup vs baseline: 7.9989x; 1.0501x over previous
import functools

import jax
import jax.numpy as jnp
import numpy as np
from jax import lax
from jax.experimental import pallas as pl
from jax.experimental.pallas import tpu as pltpu
from jax.experimental.pallas import tpu_sc as plsc

_NUM_FEATURES = 26
_ROWS_PER_FEATURE = 100000
_CHUNK = 128
_SEG = 1024


@functools.cache
def _build_transpose(dim, rows, nw):
    nseg = (rows // _SEG)
    rows_main = nseg * _SEG
    tail = rows - rows_main
    n_iter = (nseg + nw - 1) // nw
    tps = _SEG // 128
    out_rows = rows * dim // 128
    tail_out = tail * dim // 128
    mesh = plsc.VectorSubcoreMesh(core_axis_name="c", subcore_axis_name="s")

    @functools.partial(
        pl.kernel,
        out_type=jax.ShapeDtypeStruct((out_rows, 128), jnp.float32),
        mesh=mesh,
        compiler_params=pltpu.CompilerParams(
            use_tc_tiling_on_sc=True, needs_layout_passes=False
        ),
        scratch_types=[
            pltpu.VMEM((2, 2 * tps, 8, 128), jnp.float32),
            pltpu.VMEM((2, _SEG * dim // 128, 128), jnp.float32),
            pltpu.SemaphoreType.DMA,
            pltpu.SemaphoreType.DMA,
            pltpu.SemaphoreType.DMA,
        ],
    )
    def k(src_hbm, tail_hbm, dst_hbm, in_v, out_v, sem0, sem1, osem):
        wid = lax.axis_index("s") * 2 + lax.axis_index("c")
        lanes = lax.iota(jnp.int32, 16)
        rowc = lanes >> 3
        colc = (lanes & 7) * dim

        def fire(s, buf, sem):
            c0 = s * _SEG
            for h in range(2):
                for j in range(tps):
                    pltpu.async_copy(
                        src_hbm.at[h, :, pl.ds(c0 + j * 128, 128)],
                        in_v.at[buf, h * tps + j],
                        sem,
                    )

        def drain(s, buf, sem):
            c0 = s * _SEG
            for h in range(2):
                for j in range(tps):
                    pltpu.make_async_copy(
                        src_hbm.at[h, :, pl.ds(c0 + j * 128, 128)],
                        in_v.at[buf, h * tps + j],
                        sem,
                    ).wait()

        orows = _SEG * dim // 128

        def process(s, buf, sem):
            drain(s, buf, sem)

            def t_body(t, c2):
                h = t >> 3
                j = t & 7
                jb = j * 16
                for d in range(8):
                    col_idx = colc + (h * 8 + d)
                    for lb in range(8):
                        v = in_v[buf, t, d, pl.ds(lb * 16, 16)]
                        row_idx = rowc + (jb + lb * 2)
                        plsc.store_scatter(out_v.at[buf], [row_idx, col_idx], v)
                return c2

            lax.fori_loop(0, 2 * tps, t_body, 0)
            pltpu.async_copy(
                out_v.at[buf], dst_hbm.at[pl.ds(s * orows, orows)], osem
            )

        def owait(s, buf):
            pltpu.make_async_copy(
                out_v.at[buf], dst_hbm.at[pl.ds(s * orows, orows)], osem
            ).wait()

        @pl.when(wid < nseg)
        def _():
            fire(wid, 0, sem0)

        def seg_body(i, carry):
            sa = (2 * i) * nw + wid
            sb = sa + nw
            sc = sa + 2 * nw

            @pl.when(sb < nseg)
            def _():
                fire(sb, 1, sem1)

            @pl.when(sa < nseg)
            def _():
                process(sa, 0, sem0)

            @pl.when(sc < nseg)
            def _():
                fire(sc, 0, sem0)

            @pl.when(sb < nseg)
            def _():
                process(sb, 1, sem1)

            @pl.when(sa < nseg)
            def _():
                owait(sa, 0)

            @pl.when(sb < nseg)
            def _():
                owait(sb, 1)

            return carry

        lax.fori_loop(0, (n_iter + 1) // 2, seg_body, 0)

        if tail:
            @pl.when(wid == nw - 1)
            def _():
                pltpu.sync_copy(tail_hbm, out_v.at[0, pl.ds(0, tail_out)])
                pltpu.sync_copy(
                    out_v.at[0, pl.ds(0, tail_out)],
                    dst_hbm.at[pl.ds(out_rows - tail_out, tail_out)],
                )

    return k


@functools.cache
def _build_lookup(batch, feat, dim, rows, nw):
    rows_per_w = batch // nw
    n_chunks = rows_per_w // _CHUNK
    chf = _CHUNK * feat
    mesh = plsc.VectorSubcoreMesh(core_axis_name="c", subcore_axis_name="s")

    @functools.partial(
        pl.kernel,
        out_type=jax.ShapeDtypeStruct((batch, dim), jnp.float32),
        mesh=mesh,
        compiler_params=pltpu.CompilerParams(use_tc_tiling_on_sc=False),
        scratch_types=[
            pltpu.VMEM((chf,), jnp.int32),
            pltpu.VMEM((chf,), jnp.int32),
            pltpu.VMEM((chf, dim), jnp.float32),
            pltpu.VMEM((_CHUNK, dim), jnp.float32),
            pltpu.SemaphoreType.DMA,
        ],
    )
    def k(x_hbm, offs_hbm, table_hbm, out_hbm, idx_v, offs_v, rows_v, acc_v, sem):
        wid = lax.axis_index("s") * 2 + lax.axis_index("c")
        base = wid * rows_per_w
        pltpu.sync_copy(offs_hbm, offs_v)

        def chunk_body(c, carry):
            cb = base + c * _CHUNK
            pltpu.sync_copy(x_hbm.at[pl.ds(cb * feat, chf)], idx_v)

            def add_body(i, carry2):
                s = i * 16
                idx_v[pl.ds(s, 16)] = idx_v[pl.ds(s, 16)] + offs_v[pl.ds(s, 16)]
                return carry2

            lax.fori_loop(0, chf // 16, add_body, 0, unroll=8)

            pltpu.async_copy(table_hbm.at[idx_v], rows_v, sem).wait()

            def sum_rows(b, carry3):
                a = rows_v.at[b * feat][...]
                for j in range(1, feat):
                    a = a + rows_v.at[b * feat + j][...]
                acc_v.at[b][...] = a
                return carry3

            lax.fori_loop(0, _CHUNK, sum_rows, 0)
            pltpu.sync_copy(acc_v, out_hbm.at[pl.ds(cb, _CHUNK)])
            return carry

        lax.fori_loop(0, n_chunks, chunk_body, 0)

    return k


def kernel(x, table):
    batch, feat = x.shape
    rows, dim = table.shape
    info = plsc.get_sparse_core_info()
    nw = info.num_cores * info.num_subcores
    rows_main = (rows // _SEG) * _SEG
    offsets = np.arange(feat, dtype=np.int32) * _ROWS_PER_FEATURE
    offs_rep = jnp.asarray(np.tile(offsets, _CHUNK))
    x_flat = x.reshape(-1).astype(jnp.int32)
    tableT3 = table.T.reshape(2, dim // 2, rows)
    tail_packed = table[rows_main:].reshape(-1, 128)
    packed = _build_transpose(dim, rows, nw)(tableT3, tail_packed)
    table_rm = packed.reshape(rows, dim)
    return _build_lookup(batch, feat, dim, rows, nw)(x_flat, offs_rep, table_rm)
